# baseline (speedup 1.0000x reference)
import functools

import jax
import jax.numpy as jnp
from jax.experimental import pallas as pl
from jax.experimental.pallas import tpu as pltpu

_K = 8192
_D = 32
_N = 8192
_TN = 128
_COMMIT = 0.25


def _vq_block_kernel(x_ref, emb_ref, enc_ref, q_ref, loss_ref, perp_ref,
                     loss_acc, counts_acc):
    step = pl.program_id(0)
    nsteps = pl.num_programs(0)

    @pl.when(step == 0)
    def _init():
        loss_acc[...] = jnp.zeros((1, 1), jnp.float32)
        counts_acc[...] = jnp.zeros_like(counts_acc)

    x = x_ref[...]
    emb = emb_ref[...]

    x2 = jnp.sum(x * x, axis=1, keepdims=True)
    e2 = jnp.sum(emb * emb, axis=1, keepdims=True).T
    emb_bf = emb.astype(jnp.bfloat16)
    xe = jax.lax.dot_general(
        x.astype(jnp.bfloat16), emb_bf, (((1,), (1,)), ((), ())),
        preferred_element_type=jnp.float32)
    dist = x2 + e2 - 2.0 * xe

    rowmin = jnp.min(dist, axis=1, keepdims=True)
    k_iota = jax.lax.broadcasted_iota(jnp.int32, (_TN, _K), 1)
    idx = jnp.min(jnp.where(dist == rowmin, k_iota, _K),
                  axis=1, keepdims=True)

    onehot = (k_iota == idx).astype(jnp.float32)
    enc_ref[...] = onehot

    q = jax.lax.dot_general(
        onehot.astype(jnp.bfloat16), emb_bf, (((1,), (0,)), ((), ())),
        preferred_element_type=jnp.float32)
    q_ref[...] = q

    d = q - x
    loss_acc[...] += jnp.sum(d * d, keepdims=True).reshape(1, 1)
    counts_acc[...] += jnp.sum(onehot, axis=0, keepdims=True)

    @pl.when(step == nsteps - 1)
    def _finalize():
        scale = (1.0 + _COMMIT) / float(_N * _D)
        loss_ref[...] = scale * loss_acc[...]
        avg = counts_acc[...] / float(_N)
        ent = -jnp.sum(avg * jnp.log(avg + 1e-10),
                       axis=1, keepdims=True)
        perp_ref[...] = jnp.exp(ent)


@functools.partial(jax.jit, static_argnames=())
def _vq(x_flat, emb_weight):
    grid = (_N // _TN,)
    enc, q, loss, perp = pl.pallas_call(
        _vq_block_kernel,
        grid=grid,
        in_specs=[
            pl.BlockSpec((_TN, _D), lambda i: (i, 0)),
            pl.BlockSpec((_K, _D), lambda i: (0, 0)),
        ],
        out_specs=[
            pl.BlockSpec((_TN, _K), lambda i: (i, 0)),
            pl.BlockSpec((_TN, _D), lambda i: (i, 0)),
            pl.BlockSpec((1, 1), lambda i: (0, 0)),
            pl.BlockSpec((1, 1), lambda i: (0, 0)),
        ],
        out_shape=[
            jax.ShapeDtypeStruct((_N, _K), jnp.float32),
            jax.ShapeDtypeStruct((_N, _D), jnp.float32),
            jax.ShapeDtypeStruct((1, 1), jnp.float32),
            jax.ShapeDtypeStruct((1, 1), jnp.float32),
        ],
        scratch_shapes=[
            pltpu.VMEM((1, 1), jnp.float32),
            pltpu.VMEM((1, _K), jnp.float32),
        ],
    )(x_flat, emb_weight)
    return enc, q, loss, perp


def kernel(inputs, emb_weight):
    b, c, h, w = inputs.shape
    x = jnp.transpose(inputs, (0, 2, 3, 1))
    x_flat = x.reshape(-1, _D)
    enc, q, loss, perp = _vq(x_flat, emb_weight)
    quantized = q.reshape(b, h, w, c)
    quantized_bchw = jnp.transpose(quantized, (0, 3, 1, 2))
    return loss[0, 0], quantized_bchw, perp[0, 0], enc


# speedup vs baseline: 1.6405x; 1.6405x over previous
import functools

import jax
import jax.numpy as jnp
from jax.experimental import pallas as pl
from jax.experimental.pallas import tpu as pltpu

_K = 8192
_D = 32
_N = 8192
_TN = 256
_COMMIT = 0.25


def _vq_block_kernel(x_ref, emb_ref, enc_ref, q_ref, loss_ref, perp_ref,
                     loss_acc, counts_acc, e2h_acc):
    step = pl.program_id(0)
    nsteps = pl.num_programs(0)

    emb = emb_ref[...]

    @pl.when(step == 0)
    def _init():
        loss_acc[...] = jnp.zeros((1, 1), jnp.float32)
        counts_acc[...] = jnp.zeros_like(counts_acc)
        e2h_acc[...] = 0.5 * jnp.sum(emb * emb, axis=1, keepdims=True).T

    x = x_ref[...]
    emb_bf = emb.astype(jnp.bfloat16)
    xe = jax.lax.dot_general(
        x.astype(jnp.bfloat16), emb_bf, (((1,), (1,)), ((), ())),
        preferred_element_type=jnp.float32)
    score = e2h_acc[...] - xe

    rowmin = jnp.min(score, axis=1, keepdims=True)
    k_iota = jax.lax.broadcasted_iota(jnp.int32, (_TN, _K), 1)
    idx = jnp.min(jnp.where(score == rowmin, k_iota, _K),
                  axis=1, keepdims=True)

    onehot = (k_iota == idx).astype(jnp.float32)
    enc_ref[...] = onehot

    q = jax.lax.dot_general(
        onehot.astype(jnp.bfloat16), emb_bf, (((1,), (0,)), ((), ())),
        preferred_element_type=jnp.float32)
    q_ref[...] = q

    d = q - x
    loss_acc[...] += jnp.sum(d * d, keepdims=True).reshape(1, 1)
    counts_acc[...] += jnp.sum(onehot, axis=0, keepdims=True)

    @pl.when(step == nsteps - 1)
    def _finalize():
        scale = (1.0 + _COMMIT) / float(_N * _D)
        loss_ref[...] = scale * loss_acc[...]
        avg = counts_acc[...] / float(_N)
        ent = -jnp.sum(avg * jnp.log(avg + 1e-10),
                       axis=1, keepdims=True)
        perp_ref[...] = jnp.exp(ent)


@functools.partial(jax.jit, static_argnames=())
def _vq(x_flat, emb_weight):
    grid = (_N // _TN,)
    enc, q, loss, perp = pl.pallas_call(
        _vq_block_kernel,
        grid=grid,
        in_specs=[
            pl.BlockSpec((_TN, _D), lambda i: (i, 0)),
            pl.BlockSpec((_K, _D), lambda i: (0, 0)),
        ],
        out_specs=[
            pl.BlockSpec((_TN, _K), lambda i: (i, 0)),
            pl.BlockSpec((_TN, _D), lambda i: (i, 0)),
            pl.BlockSpec((1, 1), lambda i: (0, 0)),
            pl.BlockSpec((1, 1), lambda i: (0, 0)),
        ],
        out_shape=[
            jax.ShapeDtypeStruct((_N, _K), jnp.float32),
            jax.ShapeDtypeStruct((_N, _D), jnp.float32),
            jax.ShapeDtypeStruct((1, 1), jnp.float32),
            jax.ShapeDtypeStruct((1, 1), jnp.float32),
        ],
        scratch_shapes=[
            pltpu.VMEM((1, 1), jnp.float32),
            pltpu.VMEM((1, _K), jnp.float32),
            pltpu.VMEM((1, _K), jnp.float32),
        ],
    )(x_flat, emb_weight)
    return enc, q, loss, perp


def kernel(inputs, emb_weight):
    b, c, h, w = inputs.shape
    x = jnp.transpose(inputs, (0, 2, 3, 1))
    x_flat = x.reshape(-1, _D)
    enc, q, loss, perp = _vq(x_flat, emb_weight)
    quantized = q.reshape(b, h, w, c)
    quantized_bchw = jnp.transpose(quantized, (0, 3, 1, 2))
    return loss[0, 0], quantized_bchw, perp[0, 0], enc


# speedup vs baseline: 1.8624x; 1.1353x over previous
import functools

import jax
import jax.numpy as jnp
from jax.experimental import pallas as pl
from jax.experimental.pallas import tpu as pltpu

_K = 8192
_D = 32
_N = 8192
_TN = 256
_COMMIT = 0.25


def _vq_block_kernel(x_ref, emb_ref, enc_ref, q_ref, loss_ref, perp_ref,
                     loss_acc, counts_acc, e2h_acc):
    step = pl.program_id(0)
    nsteps = pl.num_programs(0)

    emb = emb_ref[...]

    @pl.when(step == 0)
    def _init():
        loss_acc[...] = jnp.zeros((1, 1), jnp.float32)
        counts_acc[...] = jnp.zeros_like(counts_acc)
        e2h_acc[...] = 0.5 * jnp.sum(emb * emb, axis=1, keepdims=True).T

    x = x_ref[...]
    emb_bf = emb.astype(jnp.bfloat16)
    xe = jax.lax.dot_general(
        x.astype(jnp.bfloat16), emb_bf, (((1,), (1,)), ((), ())),
        preferred_element_type=jnp.float32)
    score = e2h_acc[...] - xe

    idx = jnp.argmin(score, axis=1).reshape(_TN, 1)
    k_iota = jax.lax.broadcasted_iota(jnp.int32, (_TN, _K), 1)
    onehot = (k_iota == idx).astype(jnp.float32)
    enc_ref[...] = onehot

    onehot_bf = onehot.astype(jnp.bfloat16)
    q = jax.lax.dot_general(
        onehot_bf, emb_bf, (((1,), (0,)), ((), ())),
        preferred_element_type=jnp.float32)
    q_ref[...] = q

    d = q - x
    loss_acc[...] += jnp.sum(d * d, keepdims=True).reshape(1, 1)
    ones_row = jnp.ones((1, _TN), jnp.bfloat16)
    counts_acc[...] += jax.lax.dot_general(
        ones_row, onehot_bf, (((1,), (0,)), ((), ())),
        preferred_element_type=jnp.float32)

    @pl.when(step == nsteps - 1)
    def _finalize():
        scale = (1.0 + _COMMIT) / float(_N * _D)
        loss_ref[...] = scale * loss_acc[...]
        avg = counts_acc[...] / float(_N)
        ent = -jnp.sum(avg * jnp.log(avg + 1e-10),
                       axis=1, keepdims=True)
        perp_ref[...] = jnp.exp(ent)


@functools.partial(jax.jit, static_argnames=())
def _vq(x_flat, emb_weight):
    grid = (_N // _TN,)
    enc, q, loss, perp = pl.pallas_call(
        _vq_block_kernel,
        grid=grid,
        in_specs=[
            pl.BlockSpec((_TN, _D), lambda i: (i, 0)),
            pl.BlockSpec((_K, _D), lambda i: (0, 0)),
        ],
        out_specs=[
            pl.BlockSpec((_TN, _K), lambda i: (i, 0)),
            pl.BlockSpec((_TN, _D), lambda i: (i, 0)),
            pl.BlockSpec((1, 1), lambda i: (0, 0)),
            pl.BlockSpec((1, 1), lambda i: (0, 0)),
        ],
        out_shape=[
            jax.ShapeDtypeStruct((_N, _K), jnp.float32),
            jax.ShapeDtypeStruct((_N, _D), jnp.float32),
            jax.ShapeDtypeStruct((1, 1), jnp.float32),
            jax.ShapeDtypeStruct((1, 1), jnp.float32),
        ],
        scratch_shapes=[
            pltpu.VMEM((1, 1), jnp.float32),
            pltpu.VMEM((1, _K), jnp.float32),
            pltpu.VMEM((1, _K), jnp.float32),
        ],
    )(x_flat, emb_weight)
    return enc, q, loss, perp


def kernel(inputs, emb_weight):
    b, c, h, w = inputs.shape
    x = jnp.transpose(inputs, (0, 2, 3, 1))
    x_flat = x.reshape(-1, _D)
    enc, q, loss, perp = _vq(x_flat, emb_weight)
    quantized = q.reshape(b, h, w, c)
    quantized_bchw = jnp.transpose(quantized, (0, 3, 1, 2))
    return loss[0, 0], quantized_bchw, perp[0, 0], enc


# speedup vs baseline: 1.9277x; 1.0351x over previous
import functools

import jax
import jax.numpy as jnp
from jax.experimental import pallas as pl
from jax.experimental.pallas import tpu as pltpu

_K = 8192
_D = 32
_N = 8192
_TN = 256
_COMMIT = 0.25


def _vq_block_kernel(x_ref, emb_ref, enc_ref, q_ref, loss_ref, perp_ref,
                     loss_acc, counts_acc, e2h_acc):
    step = pl.program_id(0)
    nsteps = pl.num_programs(0)

    emb = emb_ref[...]

    @pl.when(step == 0)
    def _init():
        loss_acc[...] = jnp.zeros((1, 1), jnp.float32)
        counts_acc[...] = jnp.zeros_like(counts_acc)
        e2h_acc[...] = -0.5 * jnp.sum(emb * emb, axis=1, keepdims=True)

    x = x_ref[...]
    emb_bf = emb.astype(jnp.bfloat16)
    e_aug = jnp.concatenate(
        [emb_bf, e2h_acc[...].astype(jnp.bfloat16)], axis=1)
    x_aug = jnp.concatenate(
        [x.astype(jnp.bfloat16), jnp.ones((_TN, 1), jnp.bfloat16)], axis=1)
    score = jax.lax.dot_general(
        x_aug, e_aug, (((1,), (1,)), ((), ())),
        preferred_element_type=jnp.float32)

    idx = jnp.argmax(score, axis=1).reshape(_TN, 1)
    k_iota = jax.lax.broadcasted_iota(jnp.int32, (_TN, _K), 1)
    onehot = (k_iota == idx).astype(jnp.float32)
    enc_ref[...] = onehot

    onehot_bf = onehot.astype(jnp.bfloat16)
    q = jax.lax.dot_general(
        onehot_bf, emb_bf, (((1,), (0,)), ((), ())),
        preferred_element_type=jnp.float32)
    q_ref[...] = q

    d = q - x
    loss_acc[...] += jnp.sum(d * d, keepdims=True).reshape(1, 1)
    ones_row = jnp.ones((1, _TN), jnp.bfloat16)
    counts_acc[...] += jax.lax.dot_general(
        ones_row, onehot_bf, (((1,), (0,)), ((), ())),
        preferred_element_type=jnp.float32)

    @pl.when(step == nsteps - 1)
    def _finalize():
        scale = (1.0 + _COMMIT) / float(_N * _D)
        loss_ref[...] = scale * loss_acc[...]
        avg = counts_acc[...] / float(_N)
        ent = -jnp.sum(avg * jnp.log(avg + 1e-10),
                       axis=1, keepdims=True)
        perp_ref[...] = jnp.exp(ent)


@functools.partial(jax.jit, static_argnames=())
def _vq(x_flat, emb_weight):
    grid = (_N // _TN,)
    enc, q, loss, perp = pl.pallas_call(
        _vq_block_kernel,
        grid=grid,
        in_specs=[
            pl.BlockSpec((_TN, _D), lambda i: (i, 0)),
            pl.BlockSpec((_K, _D), lambda i: (0, 0)),
        ],
        out_specs=[
            pl.BlockSpec((_TN, _K), lambda i: (i, 0)),
            pl.BlockSpec((_TN, _D), lambda i: (i, 0)),
            pl.BlockSpec((1, 1), lambda i: (0, 0)),
            pl.BlockSpec((1, 1), lambda i: (0, 0)),
        ],
        out_shape=[
            jax.ShapeDtypeStruct((_N, _K), jnp.float32),
            jax.ShapeDtypeStruct((_N, _D), jnp.float32),
            jax.ShapeDtypeStruct((1, 1), jnp.float32),
            jax.ShapeDtypeStruct((1, 1), jnp.float32),
        ],
        scratch_shapes=[
            pltpu.VMEM((1, 1), jnp.float32),
            pltpu.VMEM((1, _K), jnp.float32),
            pltpu.VMEM((_K, 1), jnp.float32),
        ],
    )(x_flat, emb_weight)
    return enc, q, loss, perp


def kernel(inputs, emb_weight):
    b, c, h, w = inputs.shape
    x = jnp.transpose(inputs, (0, 2, 3, 1))
    x_flat = x.reshape(-1, _D)
    enc, q, loss, perp = _vq(x_flat, emb_weight)
    quantized = q.reshape(b, h, w, c)
    quantized_bchw = jnp.transpose(quantized, (0, 3, 1, 2))
    return loss[0, 0], quantized_bchw, perp[0, 0], enc
